# baseline (speedup 1.0000x reference)
import functools

import jax
import jax.numpy as jnp
from jax.experimental import pallas as pl
from jax.experimental.pallas import tpu as pltpu

N = 256
FD = 128
ED = 64
TI = 32
NC = N // TI


def _edge_kernel(node_ref, adj_ref, wa1_ref, ba1_ref, wa2_ref, ba2_ref,
                 w1a_ref, w1d_ref, b1_ref, g1_ref, be1_ref, w2_ref, b2_ref,
                 g2_ref, be2_ref, w3_ref, b3_ref, out_ref, nf_s, scores_s,
                 raw_s):
    s = pl.program_id(1)

    @pl.when(s == 0)
    def _precompute():
        x = node_ref[0]
        norm = jnp.sqrt(jnp.sum(x * x, axis=1, keepdims=True))
        nf_s[...] = x / jnp.maximum(norm, 1e-12)

    @pl.when((s >= 1) & (s <= NC))
    def _edge_tile():
        t = s - 1
        r0 = t * TI
        nf = nf_s[...]
        nf_i = nf_s[pl.ds(r0, TI), :]
        M = TI * N
        raw_s[:, :FD] = jnp.broadcast_to(
            nf_i.reshape(TI, 1, FD), (TI, N, FD)).reshape(M, FD)
        raw_s[:, FD:] = jnp.broadcast_to(
            nf.reshape(1, N, FD), (TI, N, FD)).reshape(M, FD)
        raw = raw_s[...]
        a = jnp.maximum(jnp.dot(raw, wa1_ref[...],
                                preferred_element_type=jnp.float32)
                        + ba1_ref[...], 0.0)
        a = jnp.dot(a, wa2_ref[...],
                    preferred_element_type=jnp.float32) + ba2_ref[...]
        amax = jnp.max(a, axis=1, keepdims=True)
        ex = jnp.exp(a - amax)
        att = ex / jnp.sum(ex, axis=1, keepdims=True)

        diff = jnp.abs(nf_i.reshape(TI, 1, FD) -
                       nf.reshape(1, N, FD)).reshape(M, FD)
        h = (jnp.dot(raw, w1a_ref[...], preferred_element_type=jnp.float32)
             + jnp.dot(diff * att, w1d_ref[...],
                       preferred_element_type=jnp.float32)) + b1_ref[...]
        h = (h / jnp.sqrt(jnp.float32(1.0 + 1e-5))) * g1_ref[...] + be1_ref[...]
        h = jnp.maximum(h, 0.0)
        h = jnp.dot(h, w2_ref[...],
                    preferred_element_type=jnp.float32) + b2_ref[...]
        h = (h / jnp.sqrt(jnp.float32(1.0 + 1e-5))) * g2_ref[...] + be2_ref[...]
        h = jnp.maximum(h, 0.0)
        logits = jnp.dot(h, w3_ref[...],
                         preferred_element_type=jnp.float32) + b3_ref[0, 0]
        scores_s[pl.ds(r0, TI), :] = jax.nn.sigmoid(logits.reshape(TI, N))

    @pl.when(s == NC + 1)
    def _select():
        scores = scores_s[...]
        mask = adj_ref[0] > 0.0
        ne = jnp.sum(mask.astype(jnp.int32))
        k = jnp.minimum(ne // 2, ne - 1)
        target = k + 1
        keys = jax.lax.bitcast_convert_type(scores, jnp.int32)
        keys = jnp.where(mask, keys, -1)

        def body(it, p):
            bit = 29 - it
            hi = p | jnp.left_shift(jnp.int32(1), bit)
            cnt = jnp.sum((keys >= hi).astype(jnp.int32))
            return jnp.where(cnt >= target, hi, p)

        p = jax.lax.fori_loop(0, 30, body, jnp.int32(0))
        out_ref[0] = jnp.where(keys >= p, scores, 0.0)


@jax.jit
def _run(node_feat, adj_matrix, wa1, ba1, wa2, ba2,
         w1a, w1d, b1, g1, be1, w2, b2, g2, be2, w3, b3):
    B = node_feat.shape[0]
    grid = (B, NC + 2)
    full = lambda b, s: (b, 0, 0)
    wspec = lambda shp: pl.BlockSpec(shp, lambda b, s: (0,) * len(shp))
    return pl.pallas_call(
        _edge_kernel,
        grid=grid,
        in_specs=[
            pl.BlockSpec((1, N, FD), full),
            pl.BlockSpec((1, N, N), full),
            wspec((2 * FD, FD)), wspec((1, FD)),
            wspec((FD, FD)), wspec((1, FD)),
            wspec((2 * FD, ED)), wspec((FD, ED)),
            wspec((1, ED)), wspec((1, ED)), wspec((1, ED)),
            wspec((ED, ED // 2)), wspec((1, ED // 2)),
            wspec((1, ED // 2)), wspec((1, ED // 2)),
            wspec((ED // 2, 1)), wspec((1, 1)),
        ],
        out_specs=pl.BlockSpec((1, N, N), full),
        out_shape=jax.ShapeDtypeStruct((B, N, N), jnp.float32),
        scratch_shapes=[
            pltpu.VMEM((N, FD), jnp.float32),
            pltpu.VMEM((N, N), jnp.float32),
            pltpu.VMEM((TI * N, 2 * FD), jnp.float32),
        ],
        compiler_params=pltpu.CompilerParams(
            dimension_semantics=("arbitrary", "arbitrary"),
        ),
    )(node_feat, adj_matrix, wa1, ba1, wa2, ba2,
      w1a, w1d, b1, g1, be1, w2, b2, g2, be2, w3, b3)


def kernel(node_feat, adj_matrix, Wa1, ba1, Wa2, ba2, W1, b1, g1, be1,
           W2, b2, g2, be2, W3, b3, current_epoch, warmup_epochs,
           temperature, graph_size_adaptation, min_edges_per_node):
    return _run(
        node_feat, adj_matrix,
        Wa1.T, ba1.reshape(1, FD),
        Wa2.T, ba2.reshape(1, FD),
        W1.T[:2 * FD], W1.T[2 * FD:],
        b1.reshape(1, ED), g1.reshape(1, ED), be1.reshape(1, ED),
        W2.T, b2.reshape(1, ED // 2), g2.reshape(1, ED // 2),
        be2.reshape(1, ED // 2),
        W3.T, b3.reshape(1, 1),
    )


# speedup vs baseline: 1.4573x; 1.4573x over previous
import jax
import jax.numpy as jnp
from jax.experimental import pallas as pl
from jax.experimental.pallas import tpu as pltpu

N = 256
FD = 128
ED = 64
TI = 32
NC = N // TI


def _edge_kernel(node_ref, adj_ref, wp1_ref, ba1_ref, wa2_ref, ba2_ref,
                 w1d_ref, b1_ref, g1_ref, be1_ref, w2_ref, b2_ref,
                 g2_ref, be2_ref, w3_ref, b3_ref, out_ref,
                 nf_s, nft_s, scores_s):
    s = pl.program_id(1)

    @pl.when(s == 0)
    def _precompute():
        x = node_ref[0]
        norm = jnp.sqrt(jnp.sum(x * x, axis=1, keepdims=True))
        nf = x / jnp.maximum(norm, 1e-12)
        nf_s[...] = nf
        nft_s[...] = nf.T

    @pl.when((s >= 1) & (s <= NC))
    def _edge_tile():
        t = s - 1
        r0 = t * TI
        M = TI * N
        nft = nft_s[...]
        nfit = nf_s[pl.ds(r0, TI), :].T
        sft = jnp.broadcast_to(nfit.reshape(FD, TI, 1),
                               (FD, TI, N)).reshape(FD, M)
        tft = jnp.broadcast_to(nft.reshape(FD, 1, N),
                               (FD, TI, N)).reshape(FD, M)
        rawt = jnp.concatenate([sft, tft], axis=0)

        p = jnp.dot(wp1_ref[...], rawt,
                    preferred_element_type=jnp.float32)
        a = jnp.maximum(p[:FD] + ba1_ref[...], 0.0)
        a = jnp.dot(wa2_ref[...], a,
                    preferred_element_type=jnp.float32) + ba2_ref[...]
        amax = jnp.max(a, axis=0, keepdims=True)
        ex = jnp.exp(a - amax)
        att = ex / jnp.sum(ex, axis=0, keepdims=True)

        datt = jnp.abs(sft - tft) * att
        h = (p[FD:] + jnp.dot(w1d_ref[...], datt,
                              preferred_element_type=jnp.float32)) + b1_ref[...]
        h = (h / jnp.sqrt(jnp.float32(1.0 + 1e-5))) * g1_ref[...] + be1_ref[...]
        h = jnp.maximum(h, 0.0)
        h = jnp.dot(w2_ref[...], h,
                    preferred_element_type=jnp.float32) + b2_ref[...]
        h = (h / jnp.sqrt(jnp.float32(1.0 + 1e-5))) * g2_ref[...] + be2_ref[...]
        h = jnp.maximum(h, 0.0)
        logits = jnp.dot(w3_ref[...], h,
                         preferred_element_type=jnp.float32) + b3_ref[0, 0]
        scores_s[pl.ds(r0, TI), :] = jax.nn.sigmoid(logits.reshape(TI, N))

    @pl.when(s == NC + 1)
    def _select():
        scores = scores_s[...]
        mask = adj_ref[0] > 0.0
        ne = jnp.sum(mask.astype(jnp.int32))
        k = jnp.minimum(ne // 2, ne - 1)
        target = k + 1
        keys = jax.lax.bitcast_convert_type(scores, jnp.int32)
        keys = jnp.where(mask, keys, -1)

        def body(it, p):
            bit = 29 - it
            hi = p | jnp.left_shift(jnp.int32(1), bit)
            cnt = jnp.sum((keys >= hi).astype(jnp.int32))
            return jnp.where(cnt >= target, hi, p)

        p = jax.lax.fori_loop(0, 30, body, jnp.int32(0))
        out_ref[0] = jnp.where(keys >= p, scores, 0.0)


@jax.jit
def _run(node_feat, adj_matrix, wp1, ba1, wa2, ba2,
         w1d, b1, g1, be1, w2, b2, g2, be2, w3, b3):
    B = node_feat.shape[0]
    grid = (B, NC + 2)
    full = lambda b, s: (b, 0, 0)
    wspec = lambda shp: pl.BlockSpec(shp, lambda b, s: (0,) * len(shp))
    return pl.pallas_call(
        _edge_kernel,
        grid=grid,
        in_specs=[
            pl.BlockSpec((1, N, FD), full),
            pl.BlockSpec((1, N, N), full),
            wspec((FD + ED, 2 * FD)), wspec((FD, 1)),
            wspec((FD, FD)), wspec((FD, 1)),
            wspec((ED, FD)), wspec((ED, 1)), wspec((ED, 1)), wspec((ED, 1)),
            wspec((ED // 2, ED)), wspec((ED // 2, 1)),
            wspec((ED // 2, 1)), wspec((ED // 2, 1)),
            wspec((1, ED // 2)), wspec((1, 1)),
        ],
        out_specs=pl.BlockSpec((1, N, N), full),
        out_shape=jax.ShapeDtypeStruct((B, N, N), jnp.float32),
        scratch_shapes=[
            pltpu.VMEM((N, FD), jnp.float32),
            pltpu.VMEM((FD, N), jnp.float32),
            pltpu.VMEM((N, N), jnp.float32),
        ],
        compiler_params=pltpu.CompilerParams(
            dimension_semantics=("arbitrary", "arbitrary"),
        ),
    )(node_feat, adj_matrix, wp1, ba1, wa2, ba2,
      w1d, b1, g1, be1, w2, b2, g2, be2, w3, b3)


def kernel(node_feat, adj_matrix, Wa1, ba1, Wa2, ba2, W1, b1, g1, be1,
           W2, b2, g2, be2, W3, b3, current_epoch, warmup_epochs,
           temperature, graph_size_adaptation, min_edges_per_node):
    wp1 = jnp.concatenate([Wa1, W1[:, :2 * FD]], axis=0)
    return _run(
        node_feat, adj_matrix,
        wp1, ba1.reshape(FD, 1),
        Wa2, ba2.reshape(FD, 1),
        W1[:, 2 * FD:], b1.reshape(ED, 1), g1.reshape(ED, 1),
        be1.reshape(ED, 1),
        W2, b2.reshape(ED // 2, 1), g2.reshape(ED // 2, 1),
        be2.reshape(ED // 2, 1),
        W3, b3.reshape(1, 1),
    )


# speedup vs baseline: 1.4579x; 1.0004x over previous
import jax
import jax.numpy as jnp
from jax.experimental import pallas as pl
from jax.experimental.pallas import tpu as pltpu

N = 256
FD = 128
ED = 64
TI = 32
NC = N // TI


def _edge_kernel(node_ref, adj_ref, wp1_ref, ba1_ref, wa2_ref, ba2_ref,
                 w1d_ref, b1_ref, g1_ref, be1_ref, w2_ref, b2_ref,
                 g2_ref, be2_ref, w3_ref, b3_ref, out_ref,
                 nf_s, nft_s, scores_s, rawt_s):
    s = pl.program_id(1)

    @pl.when(s == 0)
    def _precompute():
        x = node_ref[0]
        norm = jnp.sqrt(jnp.sum(x * x, axis=1, keepdims=True))
        nf = x / jnp.maximum(norm, 1e-12)
        nf_s[...] = nf
        nft_s[...] = nf.T

    @pl.when((s >= 1) & (s <= NC))
    def _edge_tile():
        t = s - 1
        r0 = t * TI
        M = TI * N
        nft = nft_s[...]
        nfit = nf_s[pl.ds(r0, TI), :].T
        rawt_s[:FD, :] = jnp.broadcast_to(nfit.reshape(FD, TI, 1),
                                          (FD, TI, N)).reshape(FD, M)
        rawt_s[FD:, :] = jnp.broadcast_to(nft.reshape(FD, 1, N),
                                          (FD, TI, N)).reshape(FD, M)
        rawt = rawt_s[...]

        p = jnp.dot(wp1_ref[...], rawt,
                    preferred_element_type=jnp.float32)
        a = jnp.maximum(p[:FD] + ba1_ref[...], 0.0)
        a = jnp.dot(wa2_ref[...], a,
                    preferred_element_type=jnp.float32) + ba2_ref[...]
        amax = jnp.max(a, axis=0, keepdims=True)
        ex = jnp.exp(a - amax)
        att = ex / jnp.sum(ex, axis=0, keepdims=True)

        datt = jnp.abs(rawt[:FD] - rawt[FD:]) * att
        h = (p[FD:] + jnp.dot(w1d_ref[...], datt,
                              preferred_element_type=jnp.float32)) + b1_ref[...]
        h = (h / jnp.sqrt(jnp.float32(1.0 + 1e-5))) * g1_ref[...] + be1_ref[...]
        h = jnp.maximum(h, 0.0)
        h = jnp.dot(w2_ref[...], h,
                    preferred_element_type=jnp.float32) + b2_ref[...]
        h = (h / jnp.sqrt(jnp.float32(1.0 + 1e-5))) * g2_ref[...] + be2_ref[...]
        h = jnp.maximum(h, 0.0)
        logits = jnp.dot(w3_ref[...], h,
                         preferred_element_type=jnp.float32) + b3_ref[0, 0]
        scores_s[pl.ds(r0, TI), :] = jax.nn.sigmoid(logits.reshape(TI, N))

    @pl.when(s == NC + 1)
    def _select():
        scores = scores_s[...]
        mask = adj_ref[0] > 0.0
        ne = jnp.sum(mask.astype(jnp.int32))
        k = jnp.minimum(ne // 2, ne - 1)
        target = k + 1
        keys = jax.lax.bitcast_convert_type(scores, jnp.int32)
        keys = jnp.where(mask, keys, -1)

        def body(it, p):
            bit = 29 - it
            hi = p | jnp.left_shift(jnp.int32(1), bit)
            cnt = jnp.sum((keys >= hi).astype(jnp.int32))
            return jnp.where(cnt >= target, hi, p)

        p = jax.lax.fori_loop(0, 30, body, jnp.int32(0))
        out_ref[0] = jnp.where(keys >= p, scores, 0.0)


@jax.jit
def _run(node_feat, adj_matrix, wp1, ba1, wa2, ba2,
         w1d, b1, g1, be1, w2, b2, g2, be2, w3, b3):
    B = node_feat.shape[0]
    grid = (B, NC + 2)
    full = lambda b, s: (b, 0, 0)
    wspec = lambda shp: pl.BlockSpec(shp, lambda b, s: (0,) * len(shp))
    return pl.pallas_call(
        _edge_kernel,
        grid=grid,
        in_specs=[
            pl.BlockSpec((1, N, FD), full),
            pl.BlockSpec((1, N, N), full),
            wspec((FD + ED, 2 * FD)), wspec((FD, 1)),
            wspec((FD, FD)), wspec((FD, 1)),
            wspec((ED, FD)), wspec((ED, 1)), wspec((ED, 1)), wspec((ED, 1)),
            wspec((ED // 2, ED)), wspec((ED // 2, 1)),
            wspec((ED // 2, 1)), wspec((ED // 2, 1)),
            wspec((1, ED // 2)), wspec((1, 1)),
        ],
        out_specs=pl.BlockSpec((1, N, N), full),
        out_shape=jax.ShapeDtypeStruct((B, N, N), jnp.float32),
        scratch_shapes=[
            pltpu.VMEM((N, FD), jnp.float32),
            pltpu.VMEM((FD, N), jnp.float32),
            pltpu.VMEM((N, N), jnp.float32),
            pltpu.VMEM((2 * FD, TI * N), jnp.float32),
        ],
        compiler_params=pltpu.CompilerParams(
            dimension_semantics=("arbitrary", "arbitrary"),
        ),
    )(node_feat, adj_matrix, wp1, ba1, wa2, ba2,
      w1d, b1, g1, be1, w2, b2, g2, be2, w3, b3)


def kernel(node_feat, adj_matrix, Wa1, ba1, Wa2, ba2, W1, b1, g1, be1,
           W2, b2, g2, be2, W3, b3, current_epoch, warmup_epochs,
           temperature, graph_size_adaptation, min_edges_per_node):
    wp1 = jnp.concatenate([Wa1, W1[:, :2 * FD]], axis=0)
    return _run(
        node_feat, adj_matrix,
        wp1, ba1.reshape(FD, 1),
        Wa2, ba2.reshape(FD, 1),
        W1[:, 2 * FD:], b1.reshape(ED, 1), g1.reshape(ED, 1),
        be1.reshape(ED, 1),
        W2, b2.reshape(ED // 2, 1), g2.reshape(ED // 2, 1),
        be2.reshape(ED // 2, 1),
        W3, b3.reshape(1, 1),
    )
